# baseline (speedup 1.0000x reference)
import functools

import jax
import jax.numpy as jnp
from jax import lax
from jax.experimental import pallas as pl
from jax.experimental.pallas import tpu as pltpu
from jax.experimental.pallas import tpu_sc as plsc

_NE = 1024
_ED = 64
_BETA = 0.25
_B = 8
_HW = 64
_TOK = 4096
_NTOK = _B * _HW * _HW
_NBLK = _NTOK // _TOK
_NELEM = _NTOK * _ED
_NW = 32
_TPW = _NTOK // _NW


def _vq_body(zt_ref, w_ref, zsq_ref, wsq_ref, idx_ref, loss_ref,
             perp_ref, hist_ref, sq_ref):
    step = pl.program_id(0)

    zt = zt_ref[...]
    w = w_ref[...]
    wsq = wsq_ref[0]
    zsq = zsq_ref[...]
    s2 = jax.lax.dot_general(zt, -2.0 * w, (((1,), (1,)), ((), ())),
                             preferred_element_type=jnp.float32)
    d = (zsq + wsq[None, :]) + s2
    fiota = jax.lax.broadcasted_iota(jnp.int32, (_TOK, _NE), 1).astype(jnp.float32)
    dmin = jnp.min(d, axis=1, keepdims=True)
    masked = jnp.where(d == dmin, fiota, jnp.float32(_NE))
    fidx = jnp.min(masked, axis=1, keepdims=True)
    idx = fidx[:, 0].astype(jnp.int32)
    idx_ref[0, 0, :] = idx

    oh = (masked == fidx).astype(jnp.float32)
    colsum = jnp.dot(jnp.ones((1, _TOK), jnp.float32), oh,
                     preferred_element_type=jnp.float32)[0]
    sqblk = jnp.sum(dmin)

    @pl.when(step == 0)
    def _init():
        hist_ref[0, :] = colsum
        sq_ref[0] = sqblk

    @pl.when(step > 0)
    def _acc():
        hist_ref[0, :] = hist_ref[0, :] + colsum
        sq_ref[0] = sq_ref[0] + sqblk

    @pl.when(step == _NBLK - 1)
    def _fin():
        loss = (1.0 + _BETA) * sq_ref[0] / float(_NELEM)
        loss_ref[...] = jnp.full((1, 1), loss, dtype=jnp.float32)
        e = hist_ref[0, :] / float(_NTOK)
        perp = jnp.exp(-jnp.sum(e * jnp.log(e + 1e-10)))
        perp_ref[...] = jnp.full((1, 1), perp, dtype=jnp.float32)


_SC_MESH = plsc.VectorSubcoreMesh(core_axis_name="c", subcore_axis_name="s")


_CH = 512


@functools.partial(
    pl.kernel,
    mesh=_SC_MESH,
    out_type=jax.ShapeDtypeStruct((_NTOK, 2 * _ED), jnp.float32),
    scratch_types=[
        pltpu.VMEM((_TPW,), jnp.int32),
        pltpu.VMEM((_CH, 2 * _ED), jnp.float32),
        pltpu.SemaphoreType.DMA,
    ],
)
def _gather_sc(idx_hbm, w_hbm, out_hbm, idx_v, rows_v, sem):
    wid = lax.axis_index("s") * 2 + lax.axis_index("c")
    base = wid * _TPW
    pltpu.sync_copy(idx_hbm.at[pl.ds(base, _TPW)], idx_v)
    for h in range(_TPW // _CH):
        pltpu.async_copy(w_hbm.at[idx_v.at[pl.ds(h * _CH, _CH)]],
                         rows_v, sem).wait()
        pltpu.sync_copy(rows_v, out_hbm.at[pl.ds(base + h * _CH, _CH)])


def _quantize_one(z, w):
    zt = jnp.transpose(z, (0, 2, 3, 1)).reshape(_NTOK, _ED)
    zsq = jnp.sum(zt ** 2, axis=1, keepdims=True)
    wsq = jnp.sum(w ** 2, axis=1).reshape(1, _NE)
    out = pl.pallas_call(
        _vq_body,
        grid=(_NBLK,),
        in_specs=[
            pl.BlockSpec((_TOK, _ED), lambda i: (i, 0)),
            pl.BlockSpec((_NE, _ED), lambda i: (0, 0)),
            pl.BlockSpec((_TOK, 1), lambda i: (i, 0)),
            pl.BlockSpec((1, _NE), lambda i: (0, 0)),
        ],
        out_specs=[
            pl.BlockSpec((1, 1, _TOK), lambda i: (i, 0, 0)),
            pl.BlockSpec((1, 1), lambda i: (0, 0)),
            pl.BlockSpec((1, 1), lambda i: (0, 0)),
        ],
        out_shape=[
            jax.ShapeDtypeStruct((_NBLK, 1, _TOK), jnp.int32),
            jax.ShapeDtypeStruct((1, 1), jnp.float32),
            jax.ShapeDtypeStruct((1, 1), jnp.float32),
        ],
        scratch_shapes=[
            pltpu.VMEM((1, _NE), jnp.float32),
            pltpu.SMEM((1,), jnp.float32),
        ],
    )(zt, w, zsq, wsq)
    idx, loss, perp = out
    idx = idx.reshape(_NTOK)
    wpad = jnp.concatenate([w, jnp.zeros((_NE, _ED), jnp.float32)], axis=1)
    zq_tok = _gather_sc(idx, wpad)[:, :_ED]
    zq = jnp.transpose(zq_tok.reshape(_B, _HW, _HW, _ED), (0, 3, 1, 2))
    return idx, zq, loss[0, 0], perp[0, 0]


def kernel(z0, z1, W_z, W):
    i0, zq0, l0, p0 = _quantize_one(z0, W_z)
    i1, zq1, l1, p1 = _quantize_one(z1, W)
    return (l0 + l1, zq0, zq1, (p0 + p1) / 2.0, i0, i1)


# speedup vs baseline: 1.0019x; 1.0019x over previous
import functools

import jax
import jax.numpy as jnp
from jax import lax
from jax.experimental import pallas as pl
from jax.experimental.pallas import tpu as pltpu
from jax.experimental.pallas import tpu_sc as plsc

_NE = 1024
_ED = 64
_BETA = 0.25
_B = 8
_HW = 64
_TOK = 4096
_NTOK = _B * _HW * _HW
_NBLK = _NTOK // _TOK
_NELEM = _NTOK * _ED
_NW = 32
_TPW = _NTOK // _NW


def _vq_body(z_ref, w_ref, zsq_ref, wsq_ref, idx_ref, loss_ref,
             perp_ref, hist_ref, sq_ref):
    step = pl.program_id(0)

    zt = jnp.transpose(z_ref[0].reshape(_ED, _TOK))
    w = w_ref[...]
    wsq = wsq_ref[0]
    zsq = zsq_ref[...]
    s2 = jax.lax.dot_general(zt, -2.0 * w, (((1,), (1,)), ((), ())),
                             preferred_element_type=jnp.float32)
    d = (zsq + wsq[None, :]) + s2
    fiota = jax.lax.broadcasted_iota(jnp.int32, (_TOK, _NE), 1).astype(jnp.float32)
    dmin = jnp.min(d, axis=1, keepdims=True)
    masked = jnp.where(d == dmin, fiota, jnp.float32(_NE))
    fidx = jnp.min(masked, axis=1, keepdims=True)
    idx = fidx[:, 0].astype(jnp.int32)
    idx_ref[0, 0, :] = idx

    oh = (masked == fidx).astype(jnp.float32)
    colsum = jnp.dot(jnp.ones((1, _TOK), jnp.float32), oh,
                     preferred_element_type=jnp.float32)[0]
    sqblk = jnp.sum(dmin)

    @pl.when(step == 0)
    def _init():
        hist_ref[0, :] = colsum
        sq_ref[0] = sqblk

    @pl.when(step > 0)
    def _acc():
        hist_ref[0, :] = hist_ref[0, :] + colsum
        sq_ref[0] = sq_ref[0] + sqblk

    @pl.when(step == _NBLK - 1)
    def _fin():
        loss = (1.0 + _BETA) * sq_ref[0] / float(_NELEM)
        loss_ref[...] = jnp.full((1, 1), loss, dtype=jnp.float32)
        e = hist_ref[0, :] / float(_NTOK)
        perp = jnp.exp(-jnp.sum(e * jnp.log(e + 1e-10)))
        perp_ref[...] = jnp.full((1, 1), perp, dtype=jnp.float32)


_SC_MESH = plsc.VectorSubcoreMesh(core_axis_name="c", subcore_axis_name="s")


_CH = 512


@functools.partial(
    pl.kernel,
    mesh=_SC_MESH,
    out_type=jax.ShapeDtypeStruct((_NTOK, 2 * _ED), jnp.float32),
    scratch_types=[
        pltpu.VMEM((_TPW,), jnp.int32),
        pltpu.VMEM((_CH, 2 * _ED), jnp.float32),
        pltpu.SemaphoreType.DMA,
    ],
)
def _gather_sc(idx_hbm, w_hbm, out_hbm, idx_v, rows_v, sem):
    wid = lax.axis_index("s") * 2 + lax.axis_index("c")
    base = wid * _TPW
    pltpu.sync_copy(idx_hbm.at[pl.ds(base, _TPW)], idx_v)
    for h in range(_TPW // _CH):
        pltpu.async_copy(w_hbm.at[idx_v.at[pl.ds(h * _CH, _CH)]],
                         rows_v, sem).wait()
        pltpu.sync_copy(rows_v, out_hbm.at[pl.ds(base + h * _CH, _CH)])


def _quantize_one(z, w):
    zsq = jnp.sum(jnp.transpose(z, (0, 2, 3, 1)).reshape(_NTOK, _ED) ** 2,
                  axis=1, keepdims=True)
    wsq = jnp.sum(w ** 2, axis=1).reshape(1, _NE)
    out = pl.pallas_call(
        _vq_body,
        grid=(_NBLK,),
        in_specs=[
            pl.BlockSpec((1, _ED, _HW, _HW), lambda i: (i, 0, 0, 0)),
            pl.BlockSpec((_NE, _ED), lambda i: (0, 0)),
            pl.BlockSpec((_TOK, 1), lambda i: (i, 0)),
            pl.BlockSpec((1, _NE), lambda i: (0, 0)),
        ],
        out_specs=[
            pl.BlockSpec((1, 1, _TOK), lambda i: (i, 0, 0)),
            pl.BlockSpec((1, 1), lambda i: (0, 0)),
            pl.BlockSpec((1, 1), lambda i: (0, 0)),
        ],
        out_shape=[
            jax.ShapeDtypeStruct((_NBLK, 1, _TOK), jnp.int32),
            jax.ShapeDtypeStruct((1, 1), jnp.float32),
            jax.ShapeDtypeStruct((1, 1), jnp.float32),
        ],
        scratch_shapes=[
            pltpu.VMEM((1, _NE), jnp.float32),
            pltpu.SMEM((1,), jnp.float32),
        ],
    )(z, w, zsq, wsq)
    idx, loss, perp = out
    idx = idx.reshape(_NTOK)
    wpad = jnp.concatenate([w, jnp.zeros((_NE, _ED), jnp.float32)], axis=1)
    zq_tok = _gather_sc(idx, wpad)[:, :_ED]
    zq = jnp.transpose(zq_tok.reshape(_B, _HW, _HW, _ED), (0, 3, 1, 2))
    return idx, zq, loss[0, 0], perp[0, 0]


def kernel(z0, z1, W_z, W):
    i0, zq0, l0, p0 = _quantize_one(z0, W_z)
    i1, zq1, l1, p1 = _quantize_one(z1, W)
    return (l0 + l1, zq0, zq1, (p0 + p1) / 2.0, i0, i1)
